# baseline (speedup 1.0000x reference)
import functools

import jax
import jax.numpy as jnp
from jax import lax
from jax.experimental import pallas as pl
from jax.experimental.pallas import tpu as pltpu
from jax.experimental.pallas import tpu_sc as plsc

BATCH = 64
CHANNELS = 192
H = 32
W = 32
HW = H * W

_CB = 8
_L = 16
_NC = CHANNELS // _L


_CP = 256


def _sc_build(e0h, e1h, eh, e0row_v, e1_v, ev):
    wid = lax.axis_index("s") * 2 + lax.axis_index("c")
    pltpu.sync_copy(e0h.at[wid], e0row_v)
    pltpu.sync_copy(e1h, e1_v)
    for w in range(W):
        for k in range(_NC):
            ev[w, pl.ds(k * _L, _L)] = (
                e0row_v[pl.ds(k * _L, _L)] + e1_v[w, pl.ds(k * _L, _L)]
            )
        for k in range(_NC, _CP // _L):
            ev[w, pl.ds(k * _L, _L)] = e0row_v[pl.ds(0, _L)]
    pltpu.sync_copy(ev, eh.at[pl.ds(wid * W, W)])


@functools.partial(
    pl.kernel,
    out_type=jax.ShapeDtypeStruct((HW, _CP), jnp.float32),
    mesh=plsc.VectorSubcoreMesh(core_axis_name="c", subcore_axis_name="s"),
    compiler_params=pltpu.CompilerParams(needs_layout_passes=False),
    scratch_types=[
        pltpu.VMEM((CHANNELS,), jnp.float32),
        pltpu.VMEM((W, CHANNELS), jnp.float32),
        pltpu.VMEM((W, _CP), jnp.float32),
    ],
)
def _sc_table(e0h, e1h, eh, e0row_v, e1_v, ev):
    _sc_build(e0h, e1h, eh, e0row_v, e1_v, ev)


_B1 = 32
_N1 = _B1 // _CB
_N2 = (BATCH - _B1) // _CB


def _add_body_direct(x_ref, e0_ref, e1_ref, o_ref):
    o_ref[...] = (
        x_ref[...] + e0_ref[...][None, :, None, :] + e1_ref[...][None, None, :, :]
    )


def _add_body_table(x_ref, e_ref, y_ref, o_ref):
    del y_ref
    et = e_ref[...][:, :CHANNELS].reshape(H, W, CHANNELS)
    o_ref[...] = x_ref[...] + et[None]


@jax.jit
def kernel(x, emb0, emb1):
    xt = jnp.transpose(x, (0, 2, 3, 1))
    y1 = pl.pallas_call(
        _add_body_direct,
        grid=(_N1,),
        in_specs=[
            pl.BlockSpec((_CB, H, W, CHANNELS), lambda i: (i, 0, 0, 0)),
            pl.BlockSpec((H, CHANNELS), lambda i: (0, 0)),
            pl.BlockSpec((W, CHANNELS), lambda i: (0, 0)),
        ],
        out_specs=pl.BlockSpec((_CB, H, W, CHANNELS), lambda i: (i, 0, 0, 0)),
        out_shape=jax.ShapeDtypeStruct((BATCH, H, W, CHANNELS), jnp.float32),
        compiler_params=pltpu.CompilerParams(
            dimension_semantics=("arbitrary",),
        ),
    )(xt, emb0, emb1)
    e = _sc_table(emb0, emb1)
    out_t = pl.pallas_call(
        _add_body_table,
        grid=(_N2,),
        in_specs=[
            pl.BlockSpec((_CB, H, W, CHANNELS), lambda i: (i + _N1, 0, 0, 0)),
            pl.BlockSpec((HW, _CP), lambda i: (0, 0)),
            pl.BlockSpec(memory_space=pltpu.MemorySpace.HBM),
        ],
        out_specs=pl.BlockSpec((_CB, H, W, CHANNELS), lambda i: (i + _N1, 0, 0, 0)),
        out_shape=jax.ShapeDtypeStruct((BATCH, H, W, CHANNELS), jnp.float32),
        input_output_aliases={2: 0},
        compiler_params=pltpu.CompilerParams(
            dimension_semantics=("arbitrary",),
        ),
    )(xt, e, y1)
    return jnp.transpose(out_t, (0, 3, 1, 2))


# speedup vs baseline: 1.5159x; 1.5159x over previous
import jax
import jax.numpy as jnp
from jax.experimental import pallas as pl
from jax.experimental.pallas import tpu as pltpu

BATCH = 64
CHANNELS = 192
H = 32
W = 32

_CB = 8


def _add_body(x_ref, e0_ref, e1_ref, o_ref):
    et = e0_ref[...][:, None, :] + e1_ref[...][None, :, :]
    o_ref[...] = x_ref[...] + et[None]


@jax.jit
def kernel(x, emb0, emb1):
    xt = jnp.transpose(x, (0, 2, 3, 1))
    out_t = pl.pallas_call(
        _add_body,
        grid=(BATCH // _CB,),
        in_specs=[
            pl.BlockSpec((_CB, H, W, CHANNELS), lambda i: (i, 0, 0, 0)),
            pl.BlockSpec((H, CHANNELS), lambda i: (0, 0)),
            pl.BlockSpec((W, CHANNELS), lambda i: (0, 0)),
        ],
        out_specs=pl.BlockSpec((_CB, H, W, CHANNELS), lambda i: (i, 0, 0, 0)),
        out_shape=jax.ShapeDtypeStruct((BATCH, H, W, CHANNELS), jnp.float32),
        compiler_params=pltpu.CompilerParams(
            dimension_semantics=("parallel",),
        ),
    )(xt, emb0, emb1)
    return jnp.transpose(out_t, (0, 3, 1, 2))
